# baseline (speedup 1.0000x reference)
import functools

import jax
import jax.numpy as jnp
from jax import lax
from jax.experimental import pallas as pl
from jax.experimental.pallas import tpu as pltpu
from jax.experimental.pallas import tpu_sc as plsc

_NUM_CORES = 2
_NUM_SUBCORES = 16
_LANES = 16


@functools.partial(jax.jit, static_argnums=(1, 2))
def _one_hot_sc(x_flat, d, j):
    bj = x_flat.shape[0]
    b = bj // j
    nw = _NUM_CORES * _NUM_SUBCORES
    b_per_w = b // nw
    assert b % nw == 0 and b_per_w % 2 == 0
    dj = d * j
    assert dj % _LANES == 0 and j >= _LANES
    starts = list(range(0, j - _LANES + 1, _LANES))
    if j % _LANES:
        starts.append(j - _LANES)

    mesh = plsc.VectorSubcoreMesh(
        core_axis_name="c", subcore_axis_name="s",
        num_cores=_NUM_CORES, num_subcores=_NUM_SUBCORES)

    @functools.partial(
        pl.kernel,
        mesh=mesh,
        compiler_params=pltpu.CompilerParams(needs_layout_passes=False),
        out_type=jax.ShapeDtypeStruct((b, dj), jnp.float32),
        scratch_types=[
            pltpu.VMEM((b_per_w * j,), jnp.int32),
            pltpu.VMEM((dj,), jnp.float32),
            pltpu.VMEM((dj,), jnp.float32),
            pltpu.SemaphoreType.DMA,
            pltpu.SemaphoreType.DMA,
        ],
    )
    def run(x_hbm, out_hbm, xv, slab0, slab1, sem0, sem1):
        cid = lax.axis_index("c")
        sid = lax.axis_index("s")
        wid = sid * _NUM_CORES + cid
        base = wid * b_per_w
        nwords = b_per_w * j

        pltpu.sync_copy(x_hbm.at[pl.ds(base * j, nwords)], xv)

        zf = jnp.zeros((_LANES,), jnp.float32)

        def zero_body(i, carry):
            slab0[pl.ds(i * _LANES, _LANES)] = zf
            slab1[pl.ds(i * _LANES, _LANES)] = zf
            return carry

        lax.fori_loop(0, dj // _LANES, zero_body, 0)

        lane = lax.iota(jnp.int32, _LANES)
        one_v = jnp.full((_LANES,), 1.0, jnp.float32)

        def paint(slab, row, value_vec):
            for s in starts:
                xchunk = xv[pl.ds(row * j + s, _LANES)]
                pos = xchunk * j + (lane + s)
                plsc.store_scatter(slab, [pos], value_vec)

        def fire(slab, sem, row):
            pltpu.async_copy(slab, out_hbm.at[base + row], sem)

        def drain(slab, sem, row):
            pltpu.make_async_copy(slab, out_hbm.at[base + row], sem).wait()

        paint(slab0, 0, one_v)
        fire(slab0, sem0, 0)
        paint(slab1, 1, one_v)
        fire(slab1, sem1, 1)

        def pair_body(p, carry):
            b0 = 2 * p
            for q, (slab, sem) in enumerate(((slab0, sem0), (slab1, sem1))):
                row = b0 + q
                drain(slab, sem, row - 2)
                paint(slab, row - 2, zf)
                paint(slab, row, one_v)
                fire(slab, sem, row)
            return carry

        lax.fori_loop(1, b_per_w // 2, pair_body, 0)

        drain(slab0, sem0, b_per_w - 2)
        drain(slab1, sem1, b_per_w - 1)

    return run(x_flat)


def kernel(X_in, ones):
    b, j = X_in.shape
    d = ones.shape[0]
    out = _one_hot_sc(X_in.reshape(-1), d, j)
    return out.reshape(b, d, j)


# speedup vs baseline: 1.0261x; 1.0261x over previous
import functools

import jax
import jax.numpy as jnp
from jax import lax
from jax.experimental import pallas as pl
from jax.experimental.pallas import tpu as pltpu
from jax.experimental.pallas import tpu_sc as plsc

_NUM_CORES = 2
_NUM_SUBCORES = 16
_LANES = 16
_ROWS_PER_DMA = 8


@functools.partial(jax.jit, static_argnums=(1, 2))
def _one_hot_sc(x_flat, d, j):
    bj = x_flat.shape[0]
    b = bj // j
    nw = _NUM_CORES * _NUM_SUBCORES
    b_per_w = b // nw
    rpd = _ROWS_PER_DMA
    assert b % nw == 0 and b_per_w % rpd == 0
    dj = d * j
    assert dj % _LANES == 0 and j >= _LANES
    ndma = b_per_w // rpd
    starts = list(range(0, j - _LANES + 1, _LANES))
    if j % _LANES:
        starts.append(j - _LANES)
    ncs = len(starts)
    epr = ncs * _LANES
    assert 128 % epr == 0 or epr % 128 == 0
    rows_per_iblock = max(1, 128 // epr)
    niblocks = b_per_w * epr // 128

    mesh = plsc.VectorSubcoreMesh(
        core_axis_name="c", subcore_axis_name="s",
        num_cores=_NUM_CORES, num_subcores=_NUM_SUBCORES)

    @functools.partial(
        pl.kernel,
        mesh=mesh,
        compiler_params=pltpu.CompilerParams(needs_layout_passes=False),
        out_type=jax.ShapeDtypeStruct((b * dj,), jnp.float32),
        scratch_types=[
            pltpu.VMEM((b_per_w * j,), jnp.int32),
            pltpu.VMEM((niblocks, 128), jnp.int32),
            pltpu.VMEM((128,), jnp.float32),
            pltpu.VMEM((dj,), jnp.float32),
            pltpu.VMEM_SHARED((rpd * dj,), jnp.float32),
            pltpu.SemaphoreType.DMA,
            pltpu.SemaphoreType.DMA,
        ],
    )
    def run(x_hbm, out_hbm, xv, idxv, onev, zslab, zshared, sem_z, sem_s):
        cid = lax.axis_index("c")
        sid = lax.axis_index("s")
        wid = sid * _NUM_CORES + cid
        base = wid * b_per_w

        pltpu.sync_copy(x_hbm.at[pl.ds(base * j, b_per_w * j)], xv)

        zf = jnp.zeros((_LANES,), jnp.float32)
        onef = jnp.full((_LANES,), 1.0, jnp.float32)
        lane = lax.iota(jnp.int32, _LANES)

        for c in range(128 // _LANES):
            onev[pl.ds(c * _LANES, _LANES)] = onef

        def idx_body(i, carry):
            for half in range(rows_per_iblock):
                row = i * rows_per_iblock + half
                for ci, s in enumerate(starts):
                    xchunk = xv[pl.ds(row * j + s, _LANES)]
                    pos = (base + row) * dj + xchunk * j + (lane + s)
                    col = half * epr + ci * _LANES
                    idxv[i, pl.ds(col, _LANES)] = pos
            return carry

        lax.fori_loop(0, niblocks, idx_body, 0)

        def zslab_body(i, carry):
            zslab[pl.ds(i * _LANES, _LANES)] = zf
            return carry

        lax.fori_loop(0, dj // _LANES, zslab_body, 0)

        @pl.when(sid == 0)
        def _fill_shared():
            for r in range(rpd):
                pltpu.sync_copy(zslab, zshared.at[pl.ds(r * dj, dj)])

        plsc.subcore_barrier()

        for i in range(ndma):
            off = (base + i * rpd) * dj
            pltpu.async_copy(zshared, out_hbm.at[pl.ds(off, rpd * dj)], sem_z)
        for i in range(ndma):
            off = (base + i * rpd) * dj
            pltpu.make_async_copy(
                zshared, out_hbm.at[pl.ds(off, rpd * dj)], sem_z).wait()

        def fire_body(i, carry):
            pltpu.async_copy(onev, out_hbm.at[idxv.at[i]], sem_s)
            return carry

        lax.fori_loop(0, niblocks, fire_body, 0)

        def drain_body(i, carry):
            pltpu.make_async_copy(onev, out_hbm.at[idxv.at[i]], sem_s).wait()
            return carry

        lax.fori_loop(0, niblocks, drain_body, 0)

    return run(x_flat)


def kernel(X_in, ones):
    b, j = X_in.shape
    d = ones.shape[0]
    out = _one_hot_sc(X_in.reshape(-1), d, j)
    return out.reshape(b, d, j)
